# baseline (speedup 1.0000x reference)
import functools

import jax
import jax.numpy as jnp
from jax.experimental import pallas as pl
from jax.experimental.pallas import tpu as pltpu

N = 10000
G = 128
NHID = 512
L = 3
EMB = NHID * L
T = 0.2


def _proj_loss_kernel(z_ref, zp_ref, wp1_ref, bp1_ref, wp2_ref, bp2_ref, out_ref):
    z = z_ref[...]
    zp = zp_ref[...]
    wp1 = wp1_ref[...]
    wp2 = wp2_ref[...]
    bp1 = bp1_ref[...]
    bp2 = bp2_ref[...]

    def proj(t):
        t = jnp.maximum(jnp.dot(t, wp1, preferred_element_type=jnp.float32) + bp1, 0.0)
        return jnp.dot(t, wp2, preferred_element_type=jnp.float32) + bp2

    a = proj(z)
    b = proj(zp)

    na = jnp.sqrt(jnp.sum(a * a, axis=1, keepdims=True))
    nb = jnp.sqrt(jnp.sum(b * b, axis=1, keepdims=True))
    s = jnp.dot(a, b.T, preferred_element_type=jnp.float32)
    sim = jnp.exp(s / (na * nb.T) / T)
    eye = (jax.lax.broadcasted_iota(jnp.int32, (G, G), 0)
           == jax.lax.broadcasted_iota(jnp.int32, (G, G), 1)).astype(jnp.float32)
    pos = jnp.sum(sim * eye, axis=1)
    rsum = jnp.sum(sim, axis=1)
    csum = jnp.sum(sim, axis=0)
    l1 = -jnp.mean(jnp.log(pos / (rsum - pos)))
    l2 = -jnp.mean(jnp.log(pos / (csum - pos)))
    out_ref[...] = (0.5 * (l1 + l2)).reshape(1, 1)


def _proj_loss(z, zp, params):
    out = pl.pallas_call(
        _proj_loss_kernel,
        out_shape=jax.ShapeDtypeStruct((1, 1), jnp.float32),
    )(z, zp, params['Wp1'], params['bp1'].reshape(1, EMB),
      params['Wp2'], params['bp2'].reshape(1, EMB))
    return out[0, 0]


def kernel(x, edge_index, batch, params):
    src = edge_index[0]
    dst = edge_index[1]

    def bn(h, g, b):
        m = h.mean(axis=0)
        v = jnp.maximum(h * h, 0.0).mean(axis=0) - m * m
        return g * (h - m) / jnp.sqrt(v + 1e-5) + b

    def gan(h):
        hg = h @ params['Wg']
        s_src = hg @ params['a_src']
        s_dst = hg @ params['a_dst']
        e = s_src[src] + s_dst[dst]
        e = jnp.where(e >= 0, e, 0.2 * e)
        w = jnp.exp(e)
        denom = jax.ops.segment_sum(w, dst, num_segments=N)
        num = jax.ops.segment_sum(w[:, None] * hg[src], dst, num_segments=N)
        return num / (denom[:, None] + 1e-16)

    xs = []
    xpos = []
    h = x
    for i in range(L):
        agg = jax.ops.segment_sum(h[src], dst, num_segments=N)
        h = (1.0 + params['eps%d' % i]) * h + agg
        h = h @ params['W%d' % i] + params['b%d' % i]
        h = jax.nn.relu(h)
        h = bn(h, params['gamma%d' % i], params['beta%d' % i])
        xs.append(h)
        xpos.append(gan(h))

    pool = lambda t: jax.ops.segment_sum(t, batch, num_segments=G)
    z = jnp.concatenate([pool(t) for t in xs], axis=1)
    zp = jnp.concatenate([pool(t) for t in xpos], axis=1)

    return _proj_loss(z, zp, params)


# speedup vs baseline: 1.2004x; 1.2004x over previous
import functools

import jax
import jax.numpy as jnp
from jax import lax
from jax.experimental import pallas as pl
from jax.experimental.pallas import tpu as pltpu
from jax.experimental.pallas import tpu_sc as plsc

N = 10000
G = 128
NHID = 512
L = 3
EMB = NHID * L
T = 0.2

E = 160000
_NS = 16
_EPT = E // _NS
_BLK = 80
_NBLK = _EPT // _BLK
_NP = 10240
_RPT = _NP // _NS
_ZR = 16
_CH = 128


@functools.lru_cache(maxsize=None)
def _seg_sum_rows_sc(nc):
    cpc = nc // 2
    mesh = plsc.VectorSubcoreMesh(core_axis_name="c", subcore_axis_name="s")

    @functools.partial(
        pl.kernel,
        out_type=jax.ShapeDtypeStruct((nc * _NP, _CH), jnp.float32),
        mesh=mesh,
        scratch_types=[
            pltpu.VMEM((_EPT,), jnp.int32),
            pltpu.VMEM((_BLK,), jnp.int32),
            pltpu.VMEM((_NBLK, _BLK), jnp.int32),
            pltpu.VMEM((_BLK, _CH), jnp.float32),
            pltpu.VMEM((_ZR, _CH), jnp.float32),
            pltpu.VMEM_SHARED((_NP, _CH), jnp.float32),
            pltpu.SemaphoreType.DMA,
        ],
    )
    def k(hc, src_h, dst_h, out, src_raw, idx_blk, dst_v, rows_v, zbuf, acc, sem):
        core = lax.axis_index("c")
        tid = lax.axis_index("s")
        e0 = tid * _EPT

        pltpu.sync_copy(src_h.at[pl.ds(e0, _EPT)], src_raw)
        pltpu.sync_copy(dst_h.at[tid], dst_v)

        @pl.loop(0, _ZR)
        def _(zi):
            @pl.loop(0, _CH // 16)
            def _(zj):
                zbuf[zi, pl.ds(zj * 16, 16)] = jnp.zeros((16,), jnp.float32)

        for ci in range(cpc):
            chunk = core * cpc + ci
            off = chunk * _NP

            @pl.loop(0, _RPT // _ZR)
            def _(zi):
                pltpu.sync_copy(zbuf, acc.at[pl.ds(tid * _RPT + zi * _ZR, _ZR)])
            plsc.subcore_barrier()

            @pl.loop(0, _NBLK)
            def _(b):
                @pl.loop(0, _BLK // 16)
                def _(i):
                    sl = pl.ds(i * 16, 16)
                    idx_blk[sl] = src_raw[pl.ds(b * _BLK + i * 16, 16)] + off
                pltpu.async_copy(hc.at[idx_blk], rows_v, sem).wait()
                pltpu.sync_copy(rows_v, acc.at[dst_v.at[b]], add=True)
            plsc.subcore_barrier()

            pltpu.sync_copy(acc.at[pl.ds(tid * _RPT, _RPT)],
                            out.at[pl.ds(off + tid * _RPT, _RPT)])
            plsc.subcore_barrier()

    return k


def _seg_sum_rows(h, src, dst):
    nc = h.shape[1] // _CH
    hc = h.reshape(N, nc, _CH).transpose(1, 0, 2)
    hc = jnp.pad(hc, ((0, 0), (0, _NP - N), (0, 0))).reshape(nc * _NP, _CH)
    out = _seg_sum_rows_sc(nc)(hc, src, dst.reshape(_NS, _NBLK, _BLK))
    out = out.reshape(nc, _NP, _CH)[:, :N, :]
    return out.transpose(1, 0, 2).reshape(N, nc * _CH)


def _proj_loss_kernel(z_ref, zp_ref, wp1_ref, bp1_ref, wp2_ref, bp2_ref, out_ref):
    z = z_ref[...]
    zp = zp_ref[...]
    wp1 = wp1_ref[...]
    wp2 = wp2_ref[...]
    bp1 = bp1_ref[...]
    bp2 = bp2_ref[...]

    def proj(t):
        t = jnp.maximum(jnp.dot(t, wp1, preferred_element_type=jnp.float32) + bp1, 0.0)
        return jnp.dot(t, wp2, preferred_element_type=jnp.float32) + bp2

    a = proj(z)
    b = proj(zp)

    na = jnp.sqrt(jnp.sum(a * a, axis=1, keepdims=True))
    nb = jnp.sqrt(jnp.sum(b * b, axis=1, keepdims=True))
    s = jnp.dot(a, b.T, preferred_element_type=jnp.float32)
    sim = jnp.exp(s / (na * nb.T) / T)
    eye = (jax.lax.broadcasted_iota(jnp.int32, (G, G), 0)
           == jax.lax.broadcasted_iota(jnp.int32, (G, G), 1)).astype(jnp.float32)
    pos = jnp.sum(sim * eye, axis=1)
    rsum = jnp.sum(sim, axis=1)
    csum = jnp.sum(sim, axis=0)
    l1 = -jnp.mean(jnp.log(pos / (rsum - pos)))
    l2 = -jnp.mean(jnp.log(pos / (csum - pos)))
    out_ref[...] = (0.5 * (l1 + l2)).reshape(1, 1)


def _proj_loss(z, zp, params):
    out = pl.pallas_call(
        _proj_loss_kernel,
        out_shape=jax.ShapeDtypeStruct((1, 1), jnp.float32),
    )(z, zp, params['Wp1'], params['bp1'].reshape(1, EMB),
      params['Wp2'], params['bp2'].reshape(1, EMB))
    return out[0, 0]


def kernel(x, edge_index, batch, params):
    src = edge_index[0]
    dst = edge_index[1]

    def bn(h, g, b):
        m = h.mean(axis=0)
        v = jnp.maximum(h * h, 0.0).mean(axis=0) - m * m
        return g * (h - m) / jnp.sqrt(v + 1e-5) + b

    def gan(h):
        hg = h @ params['Wg']
        s_src = hg @ params['a_src']
        s_dst = hg @ params['a_dst']
        e = s_src[src] + s_dst[dst]
        e = jnp.where(e >= 0, e, 0.2 * e)
        w = jnp.exp(e)
        denom = jax.ops.segment_sum(w, dst, num_segments=N)
        num = jax.ops.segment_sum(w[:, None] * hg[src], dst, num_segments=N)
        return num / (denom[:, None] + 1e-16)

    xs = []
    xpos = []
    h = x
    for i in range(L):
        agg = _seg_sum_rows(h, src, dst)
        h = (1.0 + params['eps%d' % i]) * h + agg
        h = h @ params['W%d' % i] + params['b%d' % i]
        h = jax.nn.relu(h)
        h = bn(h, params['gamma%d' % i], params['beta%d' % i])
        xs.append(h)
        xpos.append(gan(h))

    pool = lambda t: jax.ops.segment_sum(t, batch, num_segments=G)
    z = jnp.concatenate([pool(t) for t in xs], axis=1)
    zp = jnp.concatenate([pool(t) for t in xpos], axis=1)

    return _proj_loss(z, zp, params)


# speedup vs baseline: 2.3322x; 1.9429x over previous
import dataclasses
import functools

import jax
import jax.numpy as jnp
from jax import lax
from jax.experimental import pallas as pl
from jax.experimental.pallas import tpu as pltpu
from jax.experimental.pallas import tpu_sc as plsc

N = 10000
G = 128
NHID = 512
L = 3
EMB = NHID * L
T = 0.2

E = 160000

_SC_CP = pltpu.CompilerParams()
if "needs_layout_passes" in pltpu.CompilerParams.__dataclass_fields__:
    _SC_CP = dataclasses.replace(_SC_CP, needs_layout_passes=False)

_NS = 16
_EPT = E // _NS
_BLK = 80
_NBLK = _EPT // _BLK
_NP = 10240
_RPT = _NP // _NS
_ZR = 16
_CH = 128


@functools.lru_cache(maxsize=None)
def _seg_sum_rows_sc(nc):
    cpc = nc // 2
    mesh = plsc.VectorSubcoreMesh(core_axis_name="c", subcore_axis_name="s")

    @functools.partial(
        pl.kernel,
        out_type=jax.ShapeDtypeStruct((nc * _NP, _CH), jnp.float32),
        mesh=mesh,
        scratch_types=[
            pltpu.VMEM((_EPT,), jnp.int32),
            pltpu.VMEM((_BLK,), jnp.int32),
            pltpu.VMEM((_NBLK, _BLK), jnp.int32),
            pltpu.VMEM((_BLK, _CH), jnp.float32),
            pltpu.VMEM((_ZR, _CH), jnp.float32),
            pltpu.VMEM_SHARED((_NP, _CH), jnp.float32),
            pltpu.SemaphoreType.DMA,
        ],
    )
    def k(hc, src_h, dst_h, out, src_raw, idx_blk, dst_v, rows_v, zbuf, acc, sem):
        core = lax.axis_index("c")
        tid = lax.axis_index("s")
        e0 = tid * _EPT

        pltpu.sync_copy(src_h.at[pl.ds(e0, _EPT)], src_raw)
        pltpu.sync_copy(dst_h.at[tid], dst_v)

        @pl.loop(0, _ZR)
        def _(zi):
            @pl.loop(0, _CH // 16)
            def _(zj):
                zbuf[zi, pl.ds(zj * 16, 16)] = jnp.zeros((16,), jnp.float32)

        for ci in range(cpc):
            chunk = core * cpc + ci
            off = chunk * _NP

            @pl.loop(0, _RPT // _ZR)
            def _(zi):
                pltpu.sync_copy(zbuf, acc.at[pl.ds(tid * _RPT + zi * _ZR, _ZR)])
            plsc.subcore_barrier()

            @pl.loop(0, _NBLK)
            def _(b):
                @pl.loop(0, _BLK // 16)
                def _(i):
                    sl = pl.ds(i * 16, 16)
                    idx_blk[sl] = src_raw[pl.ds(b * _BLK + i * 16, 16)] + off
                pltpu.async_copy(hc.at[idx_blk], rows_v, sem).wait()
                pltpu.sync_copy(rows_v, acc.at[dst_v.at[b]], add=True)
            plsc.subcore_barrier()

            pltpu.sync_copy(acc.at[pl.ds(tid * _RPT, _RPT)],
                            out.at[pl.ds(off + tid * _RPT, _RPT)])
            plsc.subcore_barrier()

    return k


_EP = 163840
_EPW = _EP // 32
_BLKW = 64
_NBLKW = _EPW // _BLKW


@functools.lru_cache(maxsize=None)
def _edge_w_sc():
    mesh = plsc.VectorSubcoreMesh(core_axis_name="c", subcore_axis_name="s")

    @functools.partial(
        pl.kernel,
        out_type=jax.ShapeDtypeStruct((_EP,), jnp.float32),
        mesh=mesh,
        compiler_params=_SC_CP,
        scratch_types=[
            pltpu.VMEM((_NP,), jnp.float32),
            pltpu.VMEM((_NP,), jnp.float32),
            pltpu.VMEM((_EPW,), jnp.int32),
            pltpu.VMEM((_EPW,), jnp.int32),
            pltpu.VMEM((_EPW,), jnp.float32),
        ],
    )
    def k(ss_h, sd_h, src_h, dst_h, w_out, ss_v, sd_v, src_v, dst_v, w_v):
        core = lax.axis_index("c")
        tid = lax.axis_index("s")
        wid = tid * 2 + core
        e0 = wid * _EPW
        pltpu.sync_copy(ss_h, ss_v)
        pltpu.sync_copy(sd_h, sd_v)
        pltpu.sync_copy(src_h.at[pl.ds(e0, _EPW)], src_v)
        pltpu.sync_copy(dst_h.at[pl.ds(e0, _EPW)], dst_v)

        @pl.loop(0, _EPW // 16)
        def _(i):
            sl = pl.ds(i * 16, 16)
            e = (plsc.load_gather(ss_v, [src_v[sl]])
                 + plsc.load_gather(sd_v, [dst_v[sl]]))
            e = jnp.where(e >= 0.0, e, 0.2 * e)
            gidx = jax.lax.iota(jnp.int32, 16) + (e0 + i * 16)
            w_v[sl] = jnp.where(gidx < E, jnp.exp(e), 0.0)

        pltpu.sync_copy(w_v, w_out.at[pl.ds(e0, _EPW)])

    return k


@functools.lru_cache(maxsize=None)
def _seg_w_sum_rows_sc(nc):
    mesh = plsc.VectorSubcoreMesh(core_axis_name="c", subcore_axis_name="s")

    @functools.partial(
        pl.kernel,
        out_type=jax.ShapeDtypeStruct((2 * (nc + 1) * _NP, _CH), jnp.float32),
        mesh=mesh,
        scratch_types=[
            pltpu.VMEM((_EPW,), jnp.int32),
            pltpu.VMEM((_BLKW,), jnp.int32),
            pltpu.VMEM((_NBLKW, _BLKW), jnp.int32),
            pltpu.VMEM((_EPW,), jnp.float32),
            pltpu.VMEM((_BLKW, _CH), jnp.float32),
            pltpu.VMEM((_ZR, _CH), jnp.float32),
            pltpu.VMEM_SHARED((_NP, _CH), jnp.float32),
            pltpu.SemaphoreType.DMA,
        ],
    )
    def k(hc, src_h, dst_h, w_h, num_out,
          src_v, idx_blk, dst_v, w_v, rows_v, zbuf, acc, sem):
        core = lax.axis_index("c")
        tid = lax.axis_index("s")
        wid = core * 16 + tid
        e0 = wid * _EPW

        pltpu.sync_copy(src_h.at[pl.ds(e0, _EPW)], src_v)
        pltpu.sync_copy(dst_h.at[wid], dst_v)
        pltpu.sync_copy(w_h.at[pl.ds(e0, _EPW)], w_v)

        @pl.loop(0, _ZR)
        def _(zi):
            @pl.loop(0, _CH // 16)
            def _(zj):
                zbuf[zi, pl.ds(zj * 16, 16)] = jnp.zeros((16,), jnp.float32)

        for ci in range(nc + 1):
            off = ci * _NP
            ones_chunk = ci == nc

            @pl.loop(0, _RPT // _ZR)
            def _(zi):
                pltpu.sync_copy(zbuf, acc.at[pl.ds(tid * _RPT + zi * _ZR, _ZR)])
            plsc.subcore_barrier()

            @pl.loop(0, _NBLKW)
            def _(b):
                if not ones_chunk:
                    @pl.loop(0, _BLKW // 16)
                    def _(i):
                        sl = pl.ds(i * 16, 16)
                        idx_blk[sl] = src_v[pl.ds(b * _BLKW + i * 16, 16)] + off
                    pltpu.async_copy(hc.at[idx_blk], rows_v, sem).wait()

                @pl.loop(0, _BLKW // 16)
                def _(g):
                    w16 = w_v[pl.ds(b * _BLKW + g * 16, 16)]
                    for j2 in range(16):
                        wspl = jnp.full((16,), w16[j2], jnp.float32)
                        row = g * 16 + j2
                        for kk in range(_CH // 16):
                            sl = pl.ds(kk * 16, 16)
                            if ones_chunk:
                                rows_v[row, sl] = wspl
                            else:
                                rows_v[row, sl] = rows_v[row, sl] * wspl

                pltpu.sync_copy(rows_v, acc.at[dst_v.at[b]], add=True)
            plsc.subcore_barrier()

            pltpu.sync_copy(
                acc.at[pl.ds(tid * _RPT, _RPT)],
                num_out.at[pl.ds((core * (nc + 1) + ci) * _NP + tid * _RPT,
                                 _RPT)])
            plsc.subcore_barrier()

    return k


def _gan_sc(hg, s_src, s_dst, src, dst):
    nc = hg.shape[1] // _CH
    hc = hg.reshape(N, nc, _CH).transpose(1, 0, 2)
    hc = jnp.pad(hc, ((0, 0), (0, _NP - N), (0, 0))).reshape(nc * _NP, _CH)
    ssp = jnp.pad(s_src, (0, _NP - N))
    sdp = jnp.pad(s_dst, (0, _NP - N))
    srcp = jnp.pad(src, (0, _EP - E), constant_values=N)
    dstp = jnp.pad(dst, (0, _EP - E), constant_values=N)
    w = _edge_w_sc()(ssp, sdp, srcp, dstp)
    num_p = _seg_w_sum_rows_sc(nc)(
        hc, srcp, dstp.reshape(32, _NBLKW, _BLKW), w)
    num_p = num_p.reshape(2, nc + 1, _NP, _CH)[:, :, :N, :].sum(0)
    num = num_p[:nc].transpose(1, 0, 2).reshape(N, nc * _CH)
    denom = num_p[nc, :, 0]
    return num, denom


def _seg_sum_rows(h, src, dst):
    nc = h.shape[1] // _CH
    hc = h.reshape(N, nc, _CH).transpose(1, 0, 2)
    hc = jnp.pad(hc, ((0, 0), (0, _NP - N), (0, 0))).reshape(nc * _NP, _CH)
    out = _seg_sum_rows_sc(nc)(hc, src, dst.reshape(_NS, _NBLK, _BLK))
    out = out.reshape(nc, _NP, _CH)[:, :N, :]
    return out.transpose(1, 0, 2).reshape(N, nc * _CH)


def _proj_loss_kernel(z_ref, zp_ref, wp1_ref, bp1_ref, wp2_ref, bp2_ref, out_ref):
    z = z_ref[...]
    zp = zp_ref[...]
    wp1 = wp1_ref[...]
    wp2 = wp2_ref[...]
    bp1 = bp1_ref[...]
    bp2 = bp2_ref[...]

    def proj(t):
        t = jnp.maximum(jnp.dot(t, wp1, preferred_element_type=jnp.float32) + bp1, 0.0)
        return jnp.dot(t, wp2, preferred_element_type=jnp.float32) + bp2

    a = proj(z)
    b = proj(zp)

    na = jnp.sqrt(jnp.sum(a * a, axis=1, keepdims=True))
    nb = jnp.sqrt(jnp.sum(b * b, axis=1, keepdims=True))
    s = jnp.dot(a, b.T, preferred_element_type=jnp.float32)
    sim = jnp.exp(s / (na * nb.T) / T)
    eye = (jax.lax.broadcasted_iota(jnp.int32, (G, G), 0)
           == jax.lax.broadcasted_iota(jnp.int32, (G, G), 1)).astype(jnp.float32)
    pos = jnp.sum(sim * eye, axis=1)
    rsum = jnp.sum(sim, axis=1)
    csum = jnp.sum(sim, axis=0)
    l1 = -jnp.mean(jnp.log(pos / (rsum - pos)))
    l2 = -jnp.mean(jnp.log(pos / (csum - pos)))
    out_ref[...] = (0.5 * (l1 + l2)).reshape(1, 1)


def _proj_loss(z, zp, params):
    out = pl.pallas_call(
        _proj_loss_kernel,
        out_shape=jax.ShapeDtypeStruct((1, 1), jnp.float32),
    )(z, zp, params['Wp1'], params['bp1'].reshape(1, EMB),
      params['Wp2'], params['bp2'].reshape(1, EMB))
    return out[0, 0]


def kernel(x, edge_index, batch, params):
    src = edge_index[0]
    dst = edge_index[1]

    def bn(h, g, b):
        m = h.mean(axis=0)
        v = jnp.maximum(h * h, 0.0).mean(axis=0) - m * m
        return g * (h - m) / jnp.sqrt(v + 1e-5) + b

    def gan(h):
        hg = h @ params['Wg']
        s_src = hg @ params['a_src']
        s_dst = hg @ params['a_dst']
        num, denom = _gan_sc(hg, s_src, s_dst, src, dst)
        return num / (denom[:, None] + 1e-16)

    xs = []
    xpos = []
    h = x
    for i in range(L):
        agg = _seg_sum_rows(h, src, dst)
        h = (1.0 + params['eps%d' % i]) * h + agg
        h = h @ params['W%d' % i] + params['b%d' % i]
        h = jax.nn.relu(h)
        h = bn(h, params['gamma%d' % i], params['beta%d' % i])
        xs.append(h)
        xpos.append(gan(h))

    pool = lambda t: jax.ops.segment_sum(t, batch, num_segments=G)
    z = jnp.concatenate([pool(t) for t in xs], axis=1)
    zp = jnp.concatenate([pool(t) for t in xpos], axis=1)

    return _proj_loss(z, zp, params)


# speedup vs baseline: 2.6066x; 1.1176x over previous
import dataclasses
import functools

import jax
import jax.numpy as jnp
from jax import lax
from jax.experimental import pallas as pl
from jax.experimental.pallas import tpu as pltpu
from jax.experimental.pallas import tpu_sc as plsc

N = 10000
G = 128
NHID = 512
L = 3
EMB = NHID * L
T = 0.2

E = 160000

_SC_CP = pltpu.CompilerParams()
if "needs_layout_passes" in pltpu.CompilerParams.__dataclass_fields__:
    _SC_CP = dataclasses.replace(_SC_CP, needs_layout_passes=False)

_NS = 16
_EPT = E // _NS
_BLK = 80
_NBLK = _EPT // _BLK
_NP = 10240
_RPT = _NP // _NS
_ZR = 16
_CH = 128


@functools.lru_cache(maxsize=None)
def _seg_sum_rows_sc(nc):
    cpc = nc // 2
    mesh = plsc.VectorSubcoreMesh(core_axis_name="c", subcore_axis_name="s")

    @functools.partial(
        pl.kernel,
        out_type=jax.ShapeDtypeStruct((nc * _NP, _CH), jnp.float32),
        mesh=mesh,
        scratch_types=[
            pltpu.VMEM((_EPT,), jnp.int32),
            pltpu.VMEM((_BLK,), jnp.int32),
            pltpu.VMEM((_NBLK, _BLK), jnp.int32),
            pltpu.VMEM((_BLK, _CH), jnp.float32),
            pltpu.VMEM((_ZR, _CH), jnp.float32),
            pltpu.VMEM_SHARED((_NP, _CH), jnp.float32),
            pltpu.SemaphoreType.DMA,
        ],
    )
    def k(hc, src_h, dst_h, out, src_raw, idx_blk, dst_v, rows_v, zbuf, acc, sem):
        core = lax.axis_index("c")
        tid = lax.axis_index("s")
        e0 = tid * _EPT

        pltpu.sync_copy(src_h.at[pl.ds(e0, _EPT)], src_raw)
        pltpu.sync_copy(dst_h.at[tid], dst_v)

        @pl.loop(0, _ZR)
        def _(zi):
            @pl.loop(0, _CH // 16)
            def _(zj):
                zbuf[zi, pl.ds(zj * 16, 16)] = jnp.zeros((16,), jnp.float32)

        for ci in range(cpc):
            chunk = core * cpc + ci
            off = chunk * _NP

            @pl.loop(0, _RPT // _ZR)
            def _(zi):
                pltpu.sync_copy(zbuf, acc.at[pl.ds(tid * _RPT + zi * _ZR, _ZR)])
            plsc.subcore_barrier()

            @pl.loop(0, _NBLK)
            def _(b):
                @pl.loop(0, _BLK // 16)
                def _(i):
                    sl = pl.ds(i * 16, 16)
                    idx_blk[sl] = src_raw[pl.ds(b * _BLK + i * 16, 16)] + off
                pltpu.async_copy(hc.at[idx_blk], rows_v, sem).wait()
                pltpu.sync_copy(rows_v, acc.at[dst_v.at[b]], add=True)
            plsc.subcore_barrier()

            pltpu.sync_copy(acc.at[pl.ds(tid * _RPT, _RPT)],
                            out.at[pl.ds(off + tid * _RPT, _RPT)])
            plsc.subcore_barrier()

    return k


_EP = 163840
_EPW = _EP // 32
_BLKW = 64
_NBLKW = _EPW // _BLKW


@functools.lru_cache(maxsize=None)
def _edge_w_sc():
    mesh = plsc.VectorSubcoreMesh(core_axis_name="c", subcore_axis_name="s")

    @functools.partial(
        pl.kernel,
        out_type=jax.ShapeDtypeStruct((_EP,), jnp.float32),
        mesh=mesh,
        compiler_params=_SC_CP,
        scratch_types=[
            pltpu.VMEM((_NP,), jnp.float32),
            pltpu.VMEM((_NP,), jnp.float32),
            pltpu.VMEM((_EPW,), jnp.int32),
            pltpu.VMEM((_EPW,), jnp.int32),
            pltpu.VMEM((_EPW,), jnp.float32),
        ],
    )
    def k(ss_h, sd_h, src_h, dst_h, w_out, ss_v, sd_v, src_v, dst_v, w_v):
        core = lax.axis_index("c")
        tid = lax.axis_index("s")
        wid = tid * 2 + core
        e0 = wid * _EPW
        pltpu.sync_copy(ss_h, ss_v)
        pltpu.sync_copy(sd_h, sd_v)
        pltpu.sync_copy(src_h.at[pl.ds(e0, _EPW)], src_v)
        pltpu.sync_copy(dst_h.at[pl.ds(e0, _EPW)], dst_v)

        @pl.loop(0, _EPW // 16)
        def _(i):
            sl = pl.ds(i * 16, 16)
            e = (plsc.load_gather(ss_v, [src_v[sl]])
                 + plsc.load_gather(sd_v, [dst_v[sl]]))
            e = jnp.where(e >= 0.0, e, 0.2 * e)
            gidx = jax.lax.iota(jnp.int32, 16) + (e0 + i * 16)
            w_v[sl] = jnp.where(gidx < E, jnp.exp(e), 0.0)

        pltpu.sync_copy(w_v, w_out.at[pl.ds(e0, _EPW)])

    return k


@functools.lru_cache(maxsize=None)
def _seg_w_sum_rows_sc(nc):
    mesh = plsc.VectorSubcoreMesh(core_axis_name="c", subcore_axis_name="s")

    @functools.partial(
        pl.kernel,
        out_type=jax.ShapeDtypeStruct((2 * (nc + 1) * _NP, _CH), jnp.float32),
        mesh=mesh,
        scratch_types=[
            pltpu.VMEM((_EPW,), jnp.int32),
            pltpu.VMEM((_BLKW,), jnp.int32),
            pltpu.VMEM((_BLKW,), jnp.int32),
            pltpu.VMEM((_NBLKW, _BLKW), jnp.int32),
            pltpu.VMEM((_EPW,), jnp.float32),
            pltpu.VMEM((_BLKW, _CH), jnp.float32),
            pltpu.VMEM((_BLKW, _CH), jnp.float32),
            pltpu.VMEM((_ZR, _CH), jnp.float32),
            pltpu.VMEM_SHARED((_NP, _CH), jnp.float32),
            pltpu.SemaphoreType.DMA,
            pltpu.SemaphoreType.DMA,
            pltpu.SemaphoreType.DMA,
            pltpu.SemaphoreType.DMA,
        ],
    )
    def k(hc, src_h, dst_h, w_h, num_out,
          src_v, idx0, idx1, dst_v, w_v, rows0, rows1, zbuf, acc,
          semg0, semg1, sems0, sems1):
        core = lax.axis_index("c")
        tid = lax.axis_index("s")
        wid = core * 16 + tid
        e0 = wid * _EPW
        idxs = (idx0, idx1)
        rows = (rows0, rows1)
        semg = (semg0, semg1)
        sems = (sems0, sems1)

        pltpu.sync_copy(src_h.at[pl.ds(e0, _EPW)], src_v)
        pltpu.sync_copy(dst_h.at[wid], dst_v)
        pltpu.sync_copy(w_h.at[pl.ds(e0, _EPW)], w_v)

        @pl.loop(0, _ZR)
        def _(zi):
            @pl.loop(0, _CH // 16)
            def _(zj):
                zbuf[zi, pl.ds(zj * 16, 16)] = jnp.zeros((16,), jnp.float32)

        def issue_gather(b, lane, off):
            @pl.loop(0, _BLKW // 16)
            def _(i):
                sl = pl.ds(i * 16, 16)
                idxs[lane][sl] = src_v[pl.ds(b * _BLKW + i * 16, 16)] + off
            pltpu.async_copy(hc.at[idxs[lane]], rows[lane], semg[lane])

        def wait_gather(lane):
            pltpu.make_async_copy(hc.at[idxs[lane]], rows[lane],
                                  semg[lane]).wait()

        def issue_scatter(b, lane):
            pltpu.async_copy(rows[lane], acc.at[dst_v.at[b]], sems[lane],
                             add=True)

        def wait_scatter(b, lane):
            pltpu.make_async_copy(rows[lane], acc.at[dst_v.at[b]],
                                  sems[lane]).wait()

        def scale(b, lane, store):
            @pl.loop(0, _BLKW // 16)
            def _(g):
                w16 = w_v[pl.ds(b * _BLKW + g * 16, 16)]
                for j2 in range(16):
                    wspl = jnp.full((16,), w16[j2], jnp.float32)
                    row = g * 16 + j2
                    for kk in range(_CH // 16):
                        sl = pl.ds(kk * 16, 16)
                        if store:
                            rows[lane][row, sl] = wspl
                        else:
                            rows[lane][row, sl] = rows[lane][row, sl] * wspl

        for ci in range(nc + 1):
            off = ci * _NP
            ones_chunk = ci == nc

            @pl.loop(0, _RPT // _ZR)
            def _(zi):
                pltpu.sync_copy(zbuf, acc.at[pl.ds(tid * _RPT + zi * _ZR, _ZR)])
            plsc.subcore_barrier()

            if ones_chunk:
                @pl.loop(0, _NBLKW)
                def _(b):
                    scale(b, 0, True)
                    pltpu.sync_copy(rows0, acc.at[dst_v.at[b]], add=True)
            else:
                issue_gather(0, 0, off)

                @pl.loop(0, _NBLKW // 2)
                def _(t):
                    b0 = 2 * t
                    b1 = b0 + 1

                    @pl.when(t > 0)
                    def _():
                        wait_scatter(b1 - 2, 1)
                    issue_gather(b1, 1, off)
                    wait_gather(0)
                    scale(b0, 0, False)
                    issue_scatter(b0, 0)
                    wait_gather(1)
                    scale(b1, 1, False)
                    issue_scatter(b1, 1)

                    @pl.when(t < _NBLKW // 2 - 1)
                    def _():
                        wait_scatter(b0, 0)
                        issue_gather(b0 + 2, 0, off)

                wait_scatter(_NBLKW - 2, 0)
                wait_scatter(_NBLKW - 1, 1)
            plsc.subcore_barrier()

            pltpu.sync_copy(
                acc.at[pl.ds(tid * _RPT, _RPT)],
                num_out.at[pl.ds((core * (nc + 1) + ci) * _NP + tid * _RPT,
                                 _RPT)])
            plsc.subcore_barrier()

    return k


def _gan_sc(hg, s_src, s_dst, src, dst):
    nc = hg.shape[1] // _CH
    hc = hg.reshape(N, nc, _CH).transpose(1, 0, 2)
    hc = jnp.pad(hc, ((0, 0), (0, _NP - N), (0, 0))).reshape(nc * _NP, _CH)
    ssp = jnp.pad(s_src, (0, _NP - N))
    sdp = jnp.pad(s_dst, (0, _NP - N))
    srcp = jnp.pad(src, (0, _EP - E), constant_values=N)
    dstp = jnp.pad(dst, (0, _EP - E), constant_values=N)
    w = _edge_w_sc()(ssp, sdp, srcp, dstp)
    num_p = _seg_w_sum_rows_sc(nc)(
        hc, srcp, dstp.reshape(32, _NBLKW, _BLKW), w)
    num_p = num_p.reshape(2, nc + 1, _NP, _CH)[:, :, :N, :].sum(0)
    num = num_p[:nc].transpose(1, 0, 2).reshape(N, nc * _CH)
    denom = num_p[nc, :, 0]
    return num, denom


def _seg_sum_rows(h, src, dst):
    nc = h.shape[1] // _CH
    hc = h.reshape(N, nc, _CH).transpose(1, 0, 2)
    hc = jnp.pad(hc, ((0, 0), (0, _NP - N), (0, 0))).reshape(nc * _NP, _CH)
    out = _seg_sum_rows_sc(nc)(hc, src, dst.reshape(_NS, _NBLK, _BLK))
    out = out.reshape(nc, _NP, _CH)[:, :N, :]
    return out.transpose(1, 0, 2).reshape(N, nc * _CH)


def _proj_loss_kernel(z_ref, zp_ref, wp1_ref, bp1_ref, wp2_ref, bp2_ref, out_ref):
    z = z_ref[...]
    zp = zp_ref[...]
    wp1 = wp1_ref[...]
    wp2 = wp2_ref[...]
    bp1 = bp1_ref[...]
    bp2 = bp2_ref[...]

    def proj(t):
        t = jnp.maximum(jnp.dot(t, wp1, preferred_element_type=jnp.float32) + bp1, 0.0)
        return jnp.dot(t, wp2, preferred_element_type=jnp.float32) + bp2

    a = proj(z)
    b = proj(zp)

    na = jnp.sqrt(jnp.sum(a * a, axis=1, keepdims=True))
    nb = jnp.sqrt(jnp.sum(b * b, axis=1, keepdims=True))
    s = jnp.dot(a, b.T, preferred_element_type=jnp.float32)
    sim = jnp.exp(s / (na * nb.T) / T)
    eye = (jax.lax.broadcasted_iota(jnp.int32, (G, G), 0)
           == jax.lax.broadcasted_iota(jnp.int32, (G, G), 1)).astype(jnp.float32)
    pos = jnp.sum(sim * eye, axis=1)
    rsum = jnp.sum(sim, axis=1)
    csum = jnp.sum(sim, axis=0)
    l1 = -jnp.mean(jnp.log(pos / (rsum - pos)))
    l2 = -jnp.mean(jnp.log(pos / (csum - pos)))
    out_ref[...] = (0.5 * (l1 + l2)).reshape(1, 1)


def _proj_loss(z, zp, params):
    out = pl.pallas_call(
        _proj_loss_kernel,
        out_shape=jax.ShapeDtypeStruct((1, 1), jnp.float32),
    )(z, zp, params['Wp1'], params['bp1'].reshape(1, EMB),
      params['Wp2'], params['bp2'].reshape(1, EMB))
    return out[0, 0]


def kernel(x, edge_index, batch, params):
    src = edge_index[0]
    dst = edge_index[1]

    def bn(h, g, b):
        m = h.mean(axis=0)
        v = jnp.maximum(h * h, 0.0).mean(axis=0) - m * m
        return g * (h - m) / jnp.sqrt(v + 1e-5) + b

    def gan(h):
        hg = h @ params['Wg']
        s_src = hg @ params['a_src']
        s_dst = hg @ params['a_dst']
        num, denom = _gan_sc(hg, s_src, s_dst, src, dst)
        return num / (denom[:, None] + 1e-16)

    xs = []
    xpos = []
    h = x
    for i in range(L):
        agg = _seg_sum_rows(h, src, dst)
        h = (1.0 + params['eps%d' % i]) * h + agg
        h = h @ params['W%d' % i] + params['b%d' % i]
        h = jax.nn.relu(h)
        h = bn(h, params['gamma%d' % i], params['beta%d' % i])
        xs.append(h)
        xpos.append(gan(h))

    pool = lambda t: jax.ops.segment_sum(t, batch, num_segments=G)
    z = jnp.concatenate([pool(t) for t in xs], axis=1)
    zp = jnp.concatenate([pool(t) for t in xpos], axis=1)

    return _proj_loss(z, zp, params)


# speedup vs baseline: 2.8762x; 1.1035x over previous
import dataclasses
import functools

import jax
import jax.numpy as jnp
from jax import lax
from jax.experimental import pallas as pl
from jax.experimental.pallas import tpu as pltpu
from jax.experimental.pallas import tpu_sc as plsc

N = 10000
G = 128
NHID = 512
L = 3
EMB = NHID * L
T = 0.2

E = 160000

_SC_CP = pltpu.CompilerParams()
if "needs_layout_passes" in pltpu.CompilerParams.__dataclass_fields__:
    _SC_CP = dataclasses.replace(_SC_CP, needs_layout_passes=False)

_NS = 16
_EPT = E // _NS
_BLK = 80
_NBLK = _EPT // _BLK
_NP = 10240
_RPT = _NP // _NS
_ZR = 16
_CH = 128


@functools.lru_cache(maxsize=None)
def _seg_sum_rows_sc(nc):
    cpc = nc // 2
    mesh = plsc.VectorSubcoreMesh(core_axis_name="c", subcore_axis_name="s")

    @functools.partial(
        pl.kernel,
        out_type=jax.ShapeDtypeStruct((nc * _NP, _CH), jnp.float32),
        mesh=mesh,
        scratch_types=[
            pltpu.VMEM((_EPT,), jnp.int32),
            pltpu.VMEM((_BLK,), jnp.int32),
            pltpu.VMEM((_NBLK, _BLK), jnp.int32),
            pltpu.VMEM((_BLK, _CH), jnp.float32),
            pltpu.VMEM((_ZR, _CH), jnp.float32),
            pltpu.VMEM_SHARED((_NP, _CH), jnp.float32),
            pltpu.SemaphoreType.DMA,
        ],
    )
    def k(hc, src_h, dst_h, out, src_raw, idx_blk, dst_v, rows_v, zbuf, acc, sem):
        core = lax.axis_index("c")
        tid = lax.axis_index("s")
        e0 = tid * _EPT

        pltpu.sync_copy(src_h.at[pl.ds(e0, _EPT)], src_raw)
        pltpu.sync_copy(dst_h.at[tid], dst_v)

        @pl.loop(0, _ZR)
        def _(zi):
            @pl.loop(0, _CH // 16)
            def _(zj):
                zbuf[zi, pl.ds(zj * 16, 16)] = jnp.zeros((16,), jnp.float32)

        for ci in range(cpc):
            chunk = core * cpc + ci
            off = chunk * _NP

            @pl.loop(0, _RPT // _ZR)
            def _(zi):
                pltpu.sync_copy(zbuf, acc.at[pl.ds(tid * _RPT + zi * _ZR, _ZR)])
            plsc.subcore_barrier()

            @pl.loop(0, _NBLK)
            def _(b):
                @pl.loop(0, _BLK // 16)
                def _(i):
                    sl = pl.ds(i * 16, 16)
                    idx_blk[sl] = src_raw[pl.ds(b * _BLK + i * 16, 16)] + off
                pltpu.async_copy(hc.at[idx_blk], rows_v, sem).wait()
                pltpu.sync_copy(rows_v, acc.at[dst_v.at[b]], add=True)
            plsc.subcore_barrier()

            pltpu.sync_copy(acc.at[pl.ds(tid * _RPT, _RPT)],
                            out.at[pl.ds(off + tid * _RPT, _RPT)])
            plsc.subcore_barrier()

    return k


_EP = 163840
_EPW = _EP // 32
_BLKW = 64
_NBLKW = _EPW // _BLKW


@functools.lru_cache(maxsize=None)
def _edge_w_sc():
    mesh = plsc.VectorSubcoreMesh(core_axis_name="c", subcore_axis_name="s")

    @functools.partial(
        pl.kernel,
        out_type=jax.ShapeDtypeStruct((_EP,), jnp.float32),
        mesh=mesh,
        compiler_params=_SC_CP,
        scratch_types=[
            pltpu.VMEM((_NP,), jnp.float32),
            pltpu.VMEM((_NP,), jnp.float32),
            pltpu.VMEM((_EPW,), jnp.int32),
            pltpu.VMEM((_EPW,), jnp.int32),
            pltpu.VMEM((_EPW,), jnp.float32),
        ],
    )
    def k(ss_h, sd_h, src_h, dst_h, w_out, ss_v, sd_v, src_v, dst_v, w_v):
        core = lax.axis_index("c")
        tid = lax.axis_index("s")
        wid = tid * 2 + core
        e0 = wid * _EPW
        pltpu.sync_copy(ss_h, ss_v)
        pltpu.sync_copy(sd_h, sd_v)
        pltpu.sync_copy(src_h.at[pl.ds(e0, _EPW)], src_v)
        pltpu.sync_copy(dst_h.at[pl.ds(e0, _EPW)], dst_v)

        @pl.loop(0, _EPW // 16)
        def _(i):
            sl = pl.ds(i * 16, 16)
            e = (plsc.load_gather(ss_v, [src_v[sl]])
                 + plsc.load_gather(sd_v, [dst_v[sl]]))
            e = jnp.where(e >= 0.0, e, 0.2 * e)
            gidx = jax.lax.iota(jnp.int32, 16) + (e0 + i * 16)
            w_v[sl] = jnp.where(gidx < E, jnp.exp(e), 0.0)

        pltpu.sync_copy(w_v, w_out.at[pl.ds(e0, _EPW)])

    return k


@functools.lru_cache(maxsize=None)
def _seg_w_sum_rows_sc(nc):
    mesh = plsc.VectorSubcoreMesh(core_axis_name="c", subcore_axis_name="s")

    @functools.partial(
        pl.kernel,
        out_type=jax.ShapeDtypeStruct((2 * (nc + 1) * _NP, _CH), jnp.float32),
        mesh=mesh,
        scratch_types=[
            pltpu.VMEM((_EPW,), jnp.int32),
            pltpu.VMEM((_BLKW,), jnp.int32),
            pltpu.VMEM((_BLKW,), jnp.int32),
            pltpu.VMEM((_NBLKW, _BLKW), jnp.int32),
            pltpu.VMEM((_EPW,), jnp.float32),
            pltpu.VMEM((_BLKW, _CH), jnp.float32),
            pltpu.VMEM((_BLKW, _CH), jnp.float32),
            pltpu.VMEM((_ZR, _CH), jnp.float32),
            pltpu.VMEM_SHARED((_NP, _CH), jnp.float32),
            pltpu.SemaphoreType.DMA,
            pltpu.SemaphoreType.DMA,
            pltpu.SemaphoreType.DMA,
            pltpu.SemaphoreType.DMA,
        ],
    )
    def k(hc, src_h, dst_h, w_h, num_out,
          src_v, idx0, idx1, dst_v, w_v, rows0, rows1, zbuf, acc,
          semg0, semg1, sems0, sems1):
        core = lax.axis_index("c")
        tid = lax.axis_index("s")
        wid = core * 16 + tid
        e0 = wid * _EPW
        idxs = (idx0, idx1)
        rows = (rows0, rows1)
        semg = (semg0, semg1)
        sems = (sems0, sems1)

        pltpu.sync_copy(src_h.at[pl.ds(e0, _EPW)], src_v)
        pltpu.sync_copy(dst_h.at[wid], dst_v)
        pltpu.sync_copy(w_h.at[pl.ds(e0, _EPW)], w_v)

        @pl.loop(0, _ZR)
        def _(zi):
            @pl.loop(0, _CH // 16)
            def _(zj):
                zbuf[zi, pl.ds(zj * 16, 16)] = jnp.zeros((16,), jnp.float32)

        def issue_gather(b, lane, off):
            @pl.loop(0, _BLKW // 16)
            def _(i):
                sl = pl.ds(i * 16, 16)
                idxs[lane][sl] = src_v[pl.ds(b * _BLKW + i * 16, 16)] + off
            pltpu.async_copy(hc.at[idxs[lane]], rows[lane], semg[lane])

        def wait_gather(lane):
            pltpu.make_async_copy(hc.at[idxs[lane]], rows[lane],
                                  semg[lane]).wait()

        def issue_scatter(b, lane):
            pltpu.async_copy(rows[lane], acc.at[dst_v.at[b]], sems[lane],
                             add=True)

        def wait_scatter(b, lane):
            pltpu.make_async_copy(rows[lane], acc.at[dst_v.at[b]],
                                  sems[lane]).wait()

        def scale(b, lane, store):
            @pl.loop(0, _BLKW // 16)
            def _(g):
                w16 = w_v[pl.ds(b * _BLKW + g * 16, 16)]
                for j2 in range(16):
                    wspl = jnp.full((16,), w16[j2], jnp.float32)
                    row = g * 16 + j2
                    for kk in range(_CH // 16):
                        sl = pl.ds(kk * 16, 16)
                        if store:
                            rows[lane][row, sl] = wspl
                        else:
                            rows[lane][row, sl] = rows[lane][row, sl] * wspl

        for ci in range(nc + 1):
            off = ci * _NP
            ones_chunk = ci == nc

            @pl.loop(0, _RPT // _ZR)
            def _(zi):
                pltpu.sync_copy(zbuf, acc.at[pl.ds(tid * _RPT + zi * _ZR, _ZR)])
            plsc.subcore_barrier()

            if ones_chunk:
                @pl.loop(0, _NBLKW)
                def _(b):
                    scale(b, 0, True)
                    pltpu.sync_copy(rows0, acc.at[dst_v.at[b]], add=True)
            else:
                issue_gather(0, 0, off)

                @pl.loop(0, _NBLKW // 2)
                def _(t):
                    b0 = 2 * t
                    b1 = b0 + 1

                    @pl.when(t > 0)
                    def _():
                        wait_scatter(b1 - 2, 1)
                    issue_gather(b1, 1, off)
                    wait_gather(0)
                    scale(b0, 0, False)
                    issue_scatter(b0, 0)
                    wait_gather(1)
                    scale(b1, 1, False)
                    issue_scatter(b1, 1)

                    @pl.when(t < _NBLKW // 2 - 1)
                    def _():
                        wait_scatter(b0, 0)
                        issue_gather(b0 + 2, 0, off)

                wait_scatter(_NBLKW - 2, 0)
                wait_scatter(_NBLKW - 1, 1)
            plsc.subcore_barrier()

            pltpu.sync_copy(
                acc.at[pl.ds(tid * _RPT, _RPT)],
                num_out.at[pl.ds((core * (nc + 1) + ci) * _NP + tid * _RPT,
                                 _RPT)])
            plsc.subcore_barrier()

    return k


def _gan_sc(hg, s_src, s_dst, src, dst):
    nc = hg.shape[1] // _CH
    hc = hg.reshape(N, nc, _CH).transpose(1, 0, 2)
    hc = jnp.pad(hc, ((0, 0), (0, _NP - N), (0, 0))).reshape(nc * _NP, _CH)
    ssp = jnp.pad(s_src, (0, _NP - N))
    sdp = jnp.pad(s_dst, (0, _NP - N))
    srcp = jnp.pad(src, (0, _EP - E), constant_values=N)
    dstp = jnp.pad(dst, (0, _EP - E), constant_values=N)
    w = _edge_w_sc()(ssp, sdp, srcp, dstp)
    num_p = _seg_w_sum_rows_sc(nc)(
        hc, srcp, dstp.reshape(32, _NBLKW, _BLKW), w)
    num_p = num_p.reshape(2, nc + 1, _NP, _CH)[:, :, :N, :].sum(0)
    num = num_p[:nc].transpose(1, 0, 2).reshape(N, nc * _CH)
    denom = num_p[nc, :, 0]
    return num, denom


def _seg_sum_rows(h, src, dst):
    nc = h.shape[1] // _CH
    hc = h.reshape(N, nc, _CH).transpose(1, 0, 2)
    hc = jnp.pad(hc, ((0, 0), (0, _NP - N), (0, 0))).reshape(nc * _NP, _CH)
    out = _seg_sum_rows_sc(nc)(hc, src, dst.reshape(_NS, _NBLK, _BLK))
    out = out.reshape(nc, _NP, _CH)[:, :N, :]
    return out.transpose(1, 0, 2).reshape(N, nc * _CH)


_RB = 1000


def _dense1_kernel(h_ref, agg_ref, w_ref, b_ref, eps_ref, u_ref, s1_ref, s2_ref):
    t = (1.0 + eps_ref[0, 0]) * h_ref[...] + agg_ref[...]
    u = jnp.dot(t, w_ref[...], preferred_element_type=jnp.float32) + b_ref[...]
    u = jnp.maximum(u, 0.0)
    u_ref[...] = u

    @pl.when(pl.program_id(0) == 0)
    def _():
        s1_ref[...] = jnp.zeros_like(s1_ref)
        s2_ref[...] = jnp.zeros_like(s2_ref)
    s1_ref[...] += jnp.sum(u, axis=0, keepdims=True)
    s2_ref[...] += jnp.sum(u * u, axis=0, keepdims=True)


def _dense1(h, agg, w, b, eps):
    d = h.shape[1]
    return pl.pallas_call(
        _dense1_kernel,
        grid=(N // _RB,),
        in_specs=[
            pl.BlockSpec((_RB, d), lambda i: (i, 0)),
            pl.BlockSpec((_RB, d), lambda i: (i, 0)),
            pl.BlockSpec((d, NHID), lambda i: (0, 0)),
            pl.BlockSpec((1, NHID), lambda i: (0, 0)),
            pl.BlockSpec((1, 1), lambda i: (0, 0)),
        ],
        out_specs=[
            pl.BlockSpec((_RB, NHID), lambda i: (i, 0)),
            pl.BlockSpec((1, NHID), lambda i: (0, 0)),
            pl.BlockSpec((1, NHID), lambda i: (0, 0)),
        ],
        out_shape=[
            jax.ShapeDtypeStruct((N, NHID), jnp.float32),
            jax.ShapeDtypeStruct((1, NHID), jnp.float32),
            jax.ShapeDtypeStruct((1, NHID), jnp.float32),
        ],
    )(h, agg, w, b.reshape(1, NHID), eps.reshape(1, 1))


def _dense2_kernel(u_ref, s1_ref, s2_ref, g_ref, be_ref, wg_ref, a2_ref,
                   h4_ref, hg_ref, sd_ref):
    mean = s1_ref[...] / N
    var = s2_ref[...] / N - mean * mean
    h4 = (g_ref[...] * (u_ref[...] - mean) / jnp.sqrt(var + 1e-5)
          + be_ref[...])
    h4_ref[...] = h4
    hg = jnp.dot(h4, wg_ref[...], preferred_element_type=jnp.float32)
    hg_ref[...] = hg
    sd_ref[...] = jnp.dot(hg, a2_ref[...], preferred_element_type=jnp.float32)


def _dense2(u, s1, s2, gamma, beta, wg, a2):
    return pl.pallas_call(
        _dense2_kernel,
        grid=(N // _RB,),
        in_specs=[
            pl.BlockSpec((_RB, NHID), lambda i: (i, 0)),
            pl.BlockSpec((1, NHID), lambda i: (0, 0)),
            pl.BlockSpec((1, NHID), lambda i: (0, 0)),
            pl.BlockSpec((1, NHID), lambda i: (0, 0)),
            pl.BlockSpec((1, NHID), lambda i: (0, 0)),
            pl.BlockSpec((NHID, NHID), lambda i: (0, 0)),
            pl.BlockSpec((NHID, 128), lambda i: (0, 0)),
        ],
        out_specs=[
            pl.BlockSpec((_RB, NHID), lambda i: (i, 0)),
            pl.BlockSpec((_RB, NHID), lambda i: (i, 0)),
            pl.BlockSpec((_RB, 128), lambda i: (i, 0)),
        ],
        out_shape=[
            jax.ShapeDtypeStruct((N, NHID), jnp.float32),
            jax.ShapeDtypeStruct((N, NHID), jnp.float32),
            jax.ShapeDtypeStruct((N, 128), jnp.float32),
        ],
    )(u, s1, s2, gamma.reshape(1, NHID), beta.reshape(1, NHID), wg, a2)


def _pool_kernel(b_ref, t_ref, z_ref):
    bt = b_ref[0, 0, :]
    onehot = (jax.lax.broadcasted_iota(jnp.int32, (G, _RB), 0)
              == bt[None, :]).astype(jnp.float32)
    z = jnp.dot(onehot, t_ref[...], preferred_element_type=jnp.float32)

    @pl.when(pl.program_id(0) == 0)
    def _():
        z_ref[...] = jnp.zeros_like(z_ref)
    z_ref[...] += z


def _pool(t, batch3):
    return pl.pallas_call(
        _pool_kernel,
        grid=(N // _RB,),
        in_specs=[
            pl.BlockSpec((1, 1, _RB), lambda i: (i, 0, 0)),
            pl.BlockSpec((_RB, NHID), lambda i: (i, 0)),
        ],
        out_specs=pl.BlockSpec((G, NHID), lambda i: (0, 0)),
        out_shape=jax.ShapeDtypeStruct((G, NHID), jnp.float32),
    )(batch3, t)


def _proj_loss_kernel(z_ref, zp_ref, wp1_ref, bp1_ref, wp2_ref, bp2_ref, out_ref):
    z = z_ref[...]
    zp = zp_ref[...]
    wp1 = wp1_ref[...]
    wp2 = wp2_ref[...]
    bp1 = bp1_ref[...]
    bp2 = bp2_ref[...]

    def proj(t):
        t = jnp.maximum(jnp.dot(t, wp1, preferred_element_type=jnp.float32) + bp1, 0.0)
        return jnp.dot(t, wp2, preferred_element_type=jnp.float32) + bp2

    a = proj(z)
    b = proj(zp)

    na = jnp.sqrt(jnp.sum(a * a, axis=1, keepdims=True))
    nb = jnp.sqrt(jnp.sum(b * b, axis=1, keepdims=True))
    s = jnp.dot(a, b.T, preferred_element_type=jnp.float32)
    sim = jnp.exp(s / (na * nb.T) / T)
    eye = (jax.lax.broadcasted_iota(jnp.int32, (G, G), 0)
           == jax.lax.broadcasted_iota(jnp.int32, (G, G), 1)).astype(jnp.float32)
    pos = jnp.sum(sim * eye, axis=1)
    rsum = jnp.sum(sim, axis=1)
    csum = jnp.sum(sim, axis=0)
    l1 = -jnp.mean(jnp.log(pos / (rsum - pos)))
    l2 = -jnp.mean(jnp.log(pos / (csum - pos)))
    out_ref[...] = (0.5 * (l1 + l2)).reshape(1, 1)


def _proj_loss(z, zp, params):
    out = pl.pallas_call(
        _proj_loss_kernel,
        out_shape=jax.ShapeDtypeStruct((1, 1), jnp.float32),
    )(z, zp, params['Wp1'], params['bp1'].reshape(1, EMB),
      params['Wp2'], params['bp2'].reshape(1, EMB))
    return out[0, 0]


def kernel(x, edge_index, batch, params):
    src = edge_index[0]
    dst = edge_index[1]
    batch3 = batch.reshape(N // _RB, 1, _RB)

    zs = []
    zps = []
    h = x
    for i in range(L):
        agg = _seg_sum_rows(h, src, dst)
        u, s1, s2 = _dense1(h, agg, params['W%d' % i], params['b%d' % i],
                            params['eps%d' % i])
        a2 = jnp.zeros((NHID, 128), jnp.float32)
        a2 = a2.at[:, 0].set(params['a_src']).at[:, 1].set(params['a_dst'])
        h4, hg, sds = _dense2(u, s1, s2, params['gamma%d' % i],
                              params['beta%d' % i], params['Wg'], a2)
        num, denom = _gan_sc(hg, sds[:, 0], sds[:, 1], src, dst)
        xpos = num / (denom[:, None] + 1e-16)
        zs.append(_pool(h4, batch3))
        zps.append(_pool(xpos, batch3))
        h = h4

    z = jnp.concatenate(zs, axis=1)
    zp = jnp.concatenate(zps, axis=1)

    return _proj_loss(z, zp, params)
